# baseline (speedup 1.0000x reference)
import functools

import jax
import jax.numpy as jnp
from jax.experimental import pallas as pl
from jax.experimental.pallas import tpu as pltpu


def _se_body(x_ref, w1_ref, b1_ref, w2_ref, b2_ref, o_ref):
    x = x_ref[...]
    s = jnp.sum(x, axis=-1, dtype=jnp.float32)
    z = jnp.dot(s, w1_ref[...], preferred_element_type=jnp.float32)
    z = jnp.maximum(z + b1_ref[...], 0.0)
    a = jnp.dot(z, w2_ref[...], preferred_element_type=jnp.float32)
    g = jax.nn.sigmoid(a + b2_ref[...])
    o_ref[...] = x * g[:, :, None].astype(jnp.bfloat16)


@functools.partial(jax.jit, static_argnames=("bblk",))
def _se_run(x, w1s, b1r, w2, b2r, *, bblk):
    B, C, HW = x.shape
    Cs = w1s.shape[1]
    return pl.pallas_call(
        _se_body,
        out_shape=jax.ShapeDtypeStruct((B, C, HW), jnp.bfloat16),
        grid=(B // bblk,),
        in_specs=[
            pl.BlockSpec((bblk, C, HW), lambda b: (b, 0, 0)),
            pl.BlockSpec((C, Cs), lambda b: (0, 0)),
            pl.BlockSpec((1, Cs), lambda b: (0, 0)),
            pl.BlockSpec((Cs, C), lambda b: (0, 0)),
            pl.BlockSpec((1, C), lambda b: (0, 0)),
        ],
        out_specs=pl.BlockSpec((bblk, C, HW), lambda b: (b, 0, 0)),
        compiler_params=pltpu.CompilerParams(
            dimension_semantics=("arbitrary",),
            vmem_limit_bytes=60 << 20,
        ),
    )(x, w1s, b1r, w2, b2r)


def kernel(x, w1, b1, w2, b2):
    B, C, H, W = x.shape
    HW = H * W
    Cs = w1.shape[1]
    xb = x.reshape(B, C, HW).astype(jnp.bfloat16)
    w1s = (w1 / jnp.float32(HW)).astype(jnp.float32)
    out = _se_run(xb, w1s, b1.reshape(1, Cs), w2, b2.reshape(1, C), bblk=4)
    return out.astype(jnp.float32).reshape(B, C, H, W)


# speedup vs baseline: 1.1527x; 1.1527x over previous
import functools

import jax
import jax.numpy as jnp
from jax.experimental import pallas as pl
from jax.experimental.pallas import tpu as pltpu


def _gate_body(x_ref, w1_ref, b1_ref, w2_ref, b2_ref, g_ref):
    s = jnp.sum(x_ref[...], axis=-1, dtype=jnp.float32)
    z = jnp.dot(s, w1_ref[...], preferred_element_type=jnp.float32)
    z = jnp.maximum(z + b1_ref[...], 0.0)
    a = jnp.dot(z, w2_ref[...], preferred_element_type=jnp.float32)
    g_ref[...] = jax.nn.sigmoid(a + b2_ref[...])[:, None, :]


@functools.partial(jax.jit, static_argnames=("bblk",))
def _se_run(x, xb, w1s, b1r, w2, b2r, *, bblk):
    B, C, HW = x.shape
    Cs = w1s.shape[1]
    gates = pl.pallas_call(
        _gate_body,
        out_shape=jax.ShapeDtypeStruct((B, 1, C), jnp.float32),
        grid=(B // bblk,),
        in_specs=[
            pl.BlockSpec((bblk, C, HW), lambda b: (b, 0, 0)),
            pl.BlockSpec((C, Cs), lambda b: (0, 0)),
            pl.BlockSpec((1, Cs), lambda b: (0, 0)),
            pl.BlockSpec((Cs, C), lambda b: (0, 0)),
            pl.BlockSpec((1, C), lambda b: (0, 0)),
        ],
        out_specs=pl.BlockSpec((bblk, 1, C), lambda b: (b, 0, 0)),
        compiler_params=pltpu.CompilerParams(
            dimension_semantics=("arbitrary",),
            vmem_limit_bytes=60 << 20,
        ),
    )(xb, w1s, b1r, w2, b2r)
    return x * gates.reshape(B, C, 1)


def kernel(x, w1, b1, w2, b2):
    B, C, H, W = x.shape
    HW = H * W
    Cs = w1.shape[1]
    xf = x.reshape(B, C, HW)
    xb = xf.astype(jnp.bfloat16)
    w1s = (w1 / jnp.float32(HW)).astype(jnp.float32)
    out = _se_run(xf, xb, w1s, b1.reshape(1, Cs), w2, b2.reshape(1, C), bblk=8)
    return out.reshape(B, C, H, W)
